# baseline (speedup 1.0000x reference)
import functools

import jax
import jax.numpy as jnp
from jax import lax
from jax.experimental import pallas as pl
from jax.experimental.pallas import tpu as pltpu
from jax.experimental.pallas import tpu_sc as plsc

B = 16384
VOCAB = 1000000
EMBED = 16
PACK = 128 // EMBED
NBLK_RP = -(-VOCAB // 8192)
NLINE = NBLK_RP * (8192 // PACK)
NC, NS = 2, 16
NW = NC * NS
RPW = B // NW
CHUNK = 128
NCH = RPW // CHUNK
RB = 2048
NBLK = B // RB
CB = 8192


def _repack_body(ut_ref, it_ref, uo_ref, io_ref):
    eye = jnp.eye(EMBED, dtype=jnp.float32)
    LB = CB // PACK
    for src, dst in ((ut_ref, uo_ref), (it_ref, io_ref)):
        x = src[...]
        acc = None
        for rw in range(PACK):
            e = jnp.pad(eye, ((0, 0), (rw * EMBED, 128 - (rw + 1) * EMBED)))
            d = lax.dot_general(x[:, rw * LB:(rw + 1) * LB], e,
                                (((0,), (0,)), ((), ())),
                                precision=lax.Precision.HIGHEST,
                                preferred_element_type=jnp.float32)
            acc = d if acc is None else acc + d
        dst[...] = acc


def _tc_repack(ut_t, it_t):
    out = jax.ShapeDtypeStruct((NLINE, 128), jnp.float32)
    return pl.pallas_call(
        _repack_body,
        grid=(pl.cdiv(VOCAB, CB),),
        in_specs=[
            pl.BlockSpec((EMBED, CB), lambda g: (0, g)),
            pl.BlockSpec((EMBED, CB), lambda g: (0, g)),
        ],
        out_specs=[
            pl.BlockSpec((CB // PACK, 128), lambda g: (g, 0)),
            pl.BlockSpec((CB // PACK, 128), lambda g: (g, 0)),
        ],
        out_shape=(out, out),
    )(ut_t, it_t)


def _sc_gather(user_lines, item_lines, uidx, pidx, nidx):
    mesh = plsc.VectorSubcoreMesh(core_axis_name="c", subcore_axis_name="s")
    out = jax.ShapeDtypeStruct((B, 128), jnp.float32)

    @functools.partial(
        pl.kernel,
        out_type=(out, out, out),
        mesh=mesh,
        scratch_types=[
            pltpu.VMEM((NCH, CHUNK), jnp.int32),
            pltpu.VMEM((NCH, CHUNK), jnp.int32),
            pltpu.VMEM((NCH, CHUNK), jnp.int32),
            pltpu.VMEM((2, CHUNK, 128), jnp.float32),
            pltpu.SemaphoreType.DMA,
            pltpu.SemaphoreType.DMA,
        ],
    )
    def gather(ut, it, ui, pi, ni, uo, po, no,
               ui_v, pi_v, ni_v, rows_v, gsem, wsem):
        wid = lax.axis_index("s") * NC + lax.axis_index("c")
        base = wid * RPW
        for src, iv in ((ui, ui_v), (pi, pi_v), (ni, ni_v)):
            for j in range(NCH):
                pltpu.sync_copy(src.at[pl.ds(base + j * CHUNK, CHUNK)], iv.at[j])
        jobs = []
        for tab, iv, dst in ((ut, ui_v, uo), (it, pi_v, po), (it, ni_v, no)):
            for j in range(NCH):
                jobs.append((tab, iv.at[j], dst.at[pl.ds(base + j * CHUNK, CHUNK)]))
        gd = [None] * len(jobs)
        wd = [None] * len(jobs)
        for c, (tab, ivr, dst) in enumerate(jobs):
            if c >= 2:
                wd[c - 2].wait()
            gd[c] = pltpu.async_copy(tab.at[ivr], rows_v.at[c % 2], gsem)
            if c >= 1:
                gd[c - 1].wait()
                wd[c - 1] = pltpu.async_copy(rows_v.at[(c - 1) % 2], jobs[c - 1][2], wsem)
        last = len(jobs) - 1
        gd[last].wait()
        wd[last] = pltpu.async_copy(rows_v.at[last % 2], jobs[last][2], wsem)
        wd[last - 1].wait()
        wd[last].wait()

    return gather(user_lines, item_lines, uidx, pidx, nidx)


def _mlp_body(uid_ref, pid_ref, nid_ref, ul_ref, pl_ref, nl_ref,
              uw0, ub0, uw1, ub1, uw2, ub2, uw3, ub3,
              iw0, ib0, iw1, ib1, iw2, ib2, iw3, ib3,
              dw, db, out_ref):
    def mm(x, W):
        return jnp.dot(x, W, preferred_element_type=jnp.float32,
                       precision=lax.Precision.HIGHEST)

    lane_grp = lax.broadcasted_iota(jnp.int32, (RB, 128), 1) // EMBED
    def masked(lines_ref, id_ref):
        sub = (id_ref[...] >> 10) & (PACK - 1)
        return jnp.where(lane_grp == sub, lines_ref[...], 0.0)

    u = jnp.maximum(mm(masked(ul_ref, uid_ref), uw0[...]) + ub0[...], 0.0)
    p = jnp.maximum(mm(masked(pl_ref, pid_ref), iw0[...]) + ib0[...], 0.0)
    n = jnp.maximum(mm(masked(nl_ref, nid_ref), iw0[...]) + ib0[...], 0.0)
    for W, b in ((uw1, ub1), (uw2, ub2), (uw3, ub3)):
        u = jnp.maximum(mm(u, W[...]) + b[...], 0.0)
    for W, b in ((iw1, ib1), (iw2, ib2), (iw3, ib3)):
        Wv, bv = W[...], b[...]
        p = jnp.maximum(mm(p, Wv) + bv, 0.0)
        n = jnp.maximum(mm(n, Wv) + bv, 0.0)
    w = dw[...]
    bias = db[...]
    pv = jax.nn.sigmoid(u * p)
    nv = jax.nn.sigmoid(u * n)
    pos_l = jnp.sum(pv * w, axis=1, keepdims=True) + bias
    neg_l = jnp.sum(nv * w, axis=1, keepdims=True) + bias
    out_ref[...] = jnp.concatenate([pos_l, neg_l], axis=1)


def _tc_mlp(user, pos, neg, ul, plines, nl, weights):
    def wspec(w):
        return pl.BlockSpec(w.shape, lambda i: (0, 0))

    in_specs = [
        pl.BlockSpec((RB, 1), lambda i: (i, 0)),
        pl.BlockSpec((RB, 1), lambda i: (i, 0)),
        pl.BlockSpec((RB, 1), lambda i: (i, 0)),
        pl.BlockSpec((RB, 128), lambda i: (i, 0)),
        pl.BlockSpec((RB, 128), lambda i: (i, 0)),
        pl.BlockSpec((RB, 128), lambda i: (i, 0)),
    ] + [wspec(w) for w in weights]

    return pl.pallas_call(
        _mlp_body,
        grid=(NBLK,),
        in_specs=in_specs,
        out_specs=pl.BlockSpec((RB, 2), lambda i: (i, 0)),
        out_shape=jax.ShapeDtypeStruct((B, 2), jnp.float32),
    )(user, pos, neg, ul, plines, nl, *weights)


def kernel(user, pos, neg, user_table, item_table,
           uW0, ub0, uW1, ub1, uW2, ub2, uW3, ub3,
           iW0, ib0, iW1, ib1, iW2, ib2, iW3, ib3,
           dW, db):
    user = user.astype(jnp.int32)
    pos = pos.astype(jnp.int32)
    neg = neg.astype(jnp.int32)

    user_lines, item_lines = _tc_repack(user_table.T, item_table.T)

    def lid(r):
        return ((r >> 13) << 10) | (r & 1023)

    ul, plines, nl = _sc_gather(
        user_lines, item_lines,
        lid(user).reshape(-1), lid(pos).reshape(-1), lid(neg).reshape(-1))

    weights = (
        jnp.tile(uW0, (PACK, 1)), ub0.reshape(1, -1),
        uW1, ub1.reshape(1, -1), uW2, ub2.reshape(1, -1), uW3, ub3.reshape(1, -1),
        jnp.tile(iW0, (PACK, 1)), ib0.reshape(1, -1),
        iW1, ib1.reshape(1, -1), iW2, ib2.reshape(1, -1), iW3, ib3.reshape(1, -1),
        dW.reshape(1, -1), db.reshape(1, 1),
    )
    return _tc_mlp(user, pos, neg, ul, plines, nl, weights)


# speedup vs baseline: 1.4873x; 1.4873x over previous
import functools

import jax
import jax.numpy as jnp
from jax import lax
from jax.experimental import pallas as pl
from jax.experimental.pallas import tpu as pltpu
from jax.experimental.pallas import tpu_sc as plsc

B = 16384
VOCAB = 1000000
EMBED = 16
PACK = 128 // EMBED
CB = 8192
NBLK_RP = -(-VOCAB // CB)
NLINE = NBLK_RP * (CB // PACK)
NC, NS = 2, 16
NW = NC * NS
RPW = B // NW
CHUNK = 128
NCH = RPW // CHUNK
RB = 2048
NBLK = B // RB


def _repack_body(src, dst):
    eye = jnp.eye(EMBED, dtype=jnp.float32)
    y = lax.dot_general(src[...], eye, (((0,), (0,)), ((), ())),
                        preferred_element_type=jnp.float32)
    dst[...] = jnp.concatenate(jnp.split(y, PACK, axis=0), axis=1)


def _tc_repack(tab_t):
    return pl.pallas_call(
        _repack_body,
        grid=(NBLK_RP,),
        in_specs=[pl.BlockSpec((EMBED, CB), lambda g: (0, g))],
        out_specs=pl.BlockSpec((CB // PACK, 128), lambda g: (g, 0)),
        out_shape=jax.ShapeDtypeStruct((NLINE, 128), jnp.float32),
    )(tab_t)


def _sc_gather(lines, idxs):
    n = len(idxs)
    mesh = plsc.VectorSubcoreMesh(core_axis_name="c", subcore_axis_name="s")
    out = jax.ShapeDtypeStruct((B, 128), jnp.float32)

    @functools.partial(
        pl.kernel,
        out_type=(out,) * n,
        mesh=mesh,
        scratch_types=[pltpu.VMEM((NCH, CHUNK), jnp.int32)] * n + [
            pltpu.VMEM((2, CHUNK, 128), jnp.float32),
            pltpu.SemaphoreType.DMA,
            pltpu.SemaphoreType.DMA,
        ],
    )
    def gather(tab, *refs):
        idx_hbm = refs[:n]
        outs = refs[n:2 * n]
        idx_v = refs[2 * n:3 * n]
        rows_v, gsem, wsem = refs[3 * n:]
        wid = lax.axis_index("s") * NC + lax.axis_index("c")
        base = wid * RPW
        for src, iv in zip(idx_hbm, idx_v):
            for j in range(NCH):
                pltpu.sync_copy(src.at[pl.ds(base + j * CHUNK, CHUNK)], iv.at[j])
        jobs = []
        for iv, dst in zip(idx_v, outs):
            for j in range(NCH):
                jobs.append((iv.at[j], dst.at[pl.ds(base + j * CHUNK, CHUNK)]))
        gd = [None] * len(jobs)
        wd = [None] * len(jobs)
        for c, (ivr, dst) in enumerate(jobs):
            if c >= 2:
                wd[c - 2].wait()
            gd[c] = pltpu.async_copy(tab.at[ivr], rows_v.at[c % 2], gsem)
            if c >= 1:
                gd[c - 1].wait()
                wd[c - 1] = pltpu.async_copy(rows_v.at[(c - 1) % 2], jobs[c - 1][1], wsem)
        last = len(jobs) - 1
        gd[last].wait()
        wd[last] = pltpu.async_copy(rows_v.at[last % 2], jobs[last][1], wsem)
        wd[last - 1].wait()
        wd[last].wait()

    return gather(lines, *idxs)


def _mlp_body(uid_ref, pid_ref, nid_ref, ul_ref, pl_ref, nl_ref,
              uw0, ub0, uw1, ub1, uw2, ub2, uw3, ub3,
              iw0, ib0, iw1, ib1, iw2, ib2, iw3, ib3,
              dw, db, out_ref):
    def mm(x, W):
        return jnp.dot(x, W, preferred_element_type=jnp.float32,
                       precision=lax.Precision.HIGHEST)

    lane_grp = lax.broadcasted_iota(jnp.int32, (RB, 128), 1) // EMBED
    def masked(lines_ref, id_ref):
        sub = (id_ref[...] >> 10) & (PACK - 1)
        return jnp.where(lane_grp == sub, lines_ref[...], 0.0)

    u = jnp.maximum(mm(masked(ul_ref, uid_ref), uw0[...]) + ub0[...], 0.0)
    p = jnp.maximum(mm(masked(pl_ref, pid_ref), iw0[...]) + ib0[...], 0.0)
    n = jnp.maximum(mm(masked(nl_ref, nid_ref), iw0[...]) + ib0[...], 0.0)
    for W, b in ((uw1, ub1), (uw2, ub2), (uw3, ub3)):
        u = jnp.maximum(mm(u, W[...]) + b[...], 0.0)
    for W, b in ((iw1, ib1), (iw2, ib2), (iw3, ib3)):
        Wv, bv = W[...], b[...]
        p = jnp.maximum(mm(p, Wv) + bv, 0.0)
        n = jnp.maximum(mm(n, Wv) + bv, 0.0)
    w = dw[...]
    bias = db[...]
    pv = jax.nn.sigmoid(u * p)
    nv = jax.nn.sigmoid(u * n)
    pos_l = jnp.sum(pv * w, axis=1, keepdims=True) + bias
    neg_l = jnp.sum(nv * w, axis=1, keepdims=True) + bias
    out_ref[...] = jnp.concatenate([pos_l, neg_l], axis=1)


def _tc_mlp(user, pos, neg, ul, plines, nl, weights):
    def wspec(w):
        return pl.BlockSpec(w.shape, lambda i: (0, 0))

    in_specs = [
        pl.BlockSpec((RB, 1), lambda i: (i, 0)),
        pl.BlockSpec((RB, 1), lambda i: (i, 0)),
        pl.BlockSpec((RB, 1), lambda i: (i, 0)),
        pl.BlockSpec((RB, 128), lambda i: (i, 0)),
        pl.BlockSpec((RB, 128), lambda i: (i, 0)),
        pl.BlockSpec((RB, 128), lambda i: (i, 0)),
    ] + [wspec(w) for w in weights]

    return pl.pallas_call(
        _mlp_body,
        grid=(NBLK,),
        in_specs=in_specs,
        out_specs=pl.BlockSpec((RB, 2), lambda i: (i, 0)),
        out_shape=jax.ShapeDtypeStruct((B, 2), jnp.float32),
    )(user, pos, neg, ul, plines, nl, *weights)


def kernel(user, pos, neg, user_table, item_table,
           uW0, ub0, uW1, ub1, uW2, ub2, uW3, ub3,
           iW0, ib0, iW1, ib1, iW2, ib2, iW3, ib3,
           dW, db):
    user = user.astype(jnp.int32)
    pos = pos.astype(jnp.int32)
    neg = neg.astype(jnp.int32)

    def lid(r):
        return ((r >> 13) << 10) | (r & 1023)

    item_lines = _tc_repack(item_table.T)
    plines, nlines = _sc_gather(
        item_lines, [lid(pos).reshape(-1), lid(neg).reshape(-1)])
    user_lines = _tc_repack(user_table.T)
    (ulines,) = _sc_gather(user_lines, [lid(user).reshape(-1)])

    weights = (
        jnp.tile(uW0, (PACK, 1)), ub0.reshape(1, -1),
        uW1, ub1.reshape(1, -1), uW2, ub2.reshape(1, -1), uW3, ub3.reshape(1, -1),
        jnp.tile(iW0, (PACK, 1)), ib0.reshape(1, -1),
        iW1, ib1.reshape(1, -1), iW2, ib2.reshape(1, -1), iW3, ib3.reshape(1, -1),
        dW.reshape(1, -1), db.reshape(1, 1),
    )
    return _tc_mlp(user, pos, neg, ulines, plines, nlines, weights)


# speedup vs baseline: 2.4844x; 1.6705x over previous
import functools

import jax
import jax.numpy as jnp
from jax import lax
from jax.experimental import pallas as pl
from jax.experimental.pallas import tpu as pltpu
from jax.experimental.pallas import tpu_sc as plsc

B = 16384
VOCAB = 1000000
EMBED = 16
PACK = 128 // EMBED
CB = 8192
NBLK_RP = -(-VOCAB // CB)
NLINE = NBLK_RP * (CB // PACK)
NC, NS = 2, 16
NW = NC * NS
RPW = B // NW
CHUNK = 128
NCH = RPW // CHUNK
RB = 2048
NBLK = B // RB


def _repack_body(src, dst):
    LB = CB // PACK
    x = src[...]
    stacked = jnp.concatenate(
        [x[:, rw * LB:(rw + 1) * LB] for rw in range(PACK)], axis=0)
    dst[...] = lax.dot_general(stacked, jnp.eye(128, dtype=jnp.float32),
                               (((0,), (0,)), ((), ())),
                               precision=lax.Precision.HIGHEST,
                               preferred_element_type=jnp.float32)


def _tc_repack(tab_t):
    return pl.pallas_call(
        _repack_body,
        grid=(NBLK_RP,),
        in_specs=[pl.BlockSpec((EMBED, CB), lambda g: (0, g))],
        out_specs=pl.BlockSpec((CB // PACK, 128), lambda g: (g, 0)),
        out_shape=jax.ShapeDtypeStruct((NLINE, 128), jnp.float32),
    )(tab_t)


def _sc_gather(lines, idxs):
    n = len(idxs)
    mesh = plsc.VectorSubcoreMesh(core_axis_name="c", subcore_axis_name="s")
    out = jax.ShapeDtypeStruct((B, 128), jnp.float32)

    @functools.partial(
        pl.kernel,
        out_type=(out,) * n,
        mesh=mesh,
        scratch_types=[pltpu.VMEM((NCH, CHUNK), jnp.int32)] * n + [
            pltpu.VMEM((2, CHUNK, 128), jnp.float32),
            pltpu.SemaphoreType.DMA,
            pltpu.SemaphoreType.DMA,
        ],
    )
    def gather(tab, *refs):
        idx_hbm = refs[:n]
        outs = refs[n:2 * n]
        idx_v = refs[2 * n:3 * n]
        rows_v, gsem, wsem = refs[3 * n:]
        wid = lax.axis_index("s") * NC + lax.axis_index("c")
        base = wid * RPW
        for src, iv in zip(idx_hbm, idx_v):
            for j in range(NCH):
                pltpu.sync_copy(src.at[pl.ds(base + j * CHUNK, CHUNK)], iv.at[j])
        jobs = []
        for iv, dst in zip(idx_v, outs):
            for j in range(NCH):
                jobs.append((iv.at[j], dst.at[pl.ds(base + j * CHUNK, CHUNK)]))
        gd = [None] * len(jobs)
        wd = [None] * len(jobs)
        for c, (ivr, dst) in enumerate(jobs):
            if c >= 2:
                wd[c - 2].wait()
            gd[c] = pltpu.async_copy(tab.at[ivr], rows_v.at[c % 2], gsem)
            if c >= 1:
                gd[c - 1].wait()
                wd[c - 1] = pltpu.async_copy(rows_v.at[(c - 1) % 2], jobs[c - 1][1], wsem)
        last = len(jobs) - 1
        gd[last].wait()
        wd[last] = pltpu.async_copy(rows_v.at[last % 2], jobs[last][1], wsem)
        wd[last - 1].wait()
        wd[last].wait()

    return gather(lines, *idxs)


def _mlp_body(uid_ref, pid_ref, nid_ref, ul_ref, pl_ref, nl_ref,
              uw0, ub0, uw1, ub1, uw2, ub2, uw3, ub3,
              iw0, ib0, iw1, ib1, iw2, ib2, iw3, ib3,
              dw, db, out_ref):
    def mm(x, W):
        return jnp.dot(x, W, preferred_element_type=jnp.float32,
                       precision=lax.Precision.HIGHEST)

    lane_grp = lax.broadcasted_iota(jnp.int32, (RB, 128), 1) // EMBED
    def masked(lines_ref, id_ref):
        sub = (id_ref[...] >> 10) & (PACK - 1)
        return jnp.where(lane_grp == sub, lines_ref[...], 0.0)

    u = jnp.maximum(mm(masked(ul_ref, uid_ref), uw0[...]) + ub0[...], 0.0)
    p = jnp.maximum(mm(masked(pl_ref, pid_ref), iw0[...]) + ib0[...], 0.0)
    n = jnp.maximum(mm(masked(nl_ref, nid_ref), iw0[...]) + ib0[...], 0.0)
    for W, b in ((uw1, ub1), (uw2, ub2), (uw3, ub3)):
        u = jnp.maximum(mm(u, W[...]) + b[...], 0.0)
    for W, b in ((iw1, ib1), (iw2, ib2), (iw3, ib3)):
        Wv, bv = W[...], b[...]
        p = jnp.maximum(mm(p, Wv) + bv, 0.0)
        n = jnp.maximum(mm(n, Wv) + bv, 0.0)
    w = dw[...]
    bias = db[...]
    pv = jax.nn.sigmoid(u * p)
    nv = jax.nn.sigmoid(u * n)
    pos_l = jnp.sum(pv * w, axis=1, keepdims=True) + bias
    neg_l = jnp.sum(nv * w, axis=1, keepdims=True) + bias
    out_ref[...] = jnp.concatenate([pos_l, neg_l], axis=1)


def _tc_mlp(user, pos, neg, ul, plines, nl, weights):
    def wspec(w):
        return pl.BlockSpec(w.shape, lambda i: (0, 0))

    in_specs = [
        pl.BlockSpec((RB, 1), lambda i: (i, 0)),
        pl.BlockSpec((RB, 1), lambda i: (i, 0)),
        pl.BlockSpec((RB, 1), lambda i: (i, 0)),
        pl.BlockSpec((RB, 128), lambda i: (i, 0)),
        pl.BlockSpec((RB, 128), lambda i: (i, 0)),
        pl.BlockSpec((RB, 128), lambda i: (i, 0)),
    ] + [wspec(w) for w in weights]

    return pl.pallas_call(
        _mlp_body,
        grid=(NBLK,),
        in_specs=in_specs,
        out_specs=pl.BlockSpec((RB, 2), lambda i: (i, 0)),
        out_shape=jax.ShapeDtypeStruct((B, 2), jnp.float32),
    )(user, pos, neg, ul, plines, nl, *weights)


def kernel(user, pos, neg, user_table, item_table,
           uW0, ub0, uW1, ub1, uW2, ub2, uW3, ub3,
           iW0, ib0, iW1, ib1, iW2, ib2, iW3, ib3,
           dW, db):
    user = user.astype(jnp.int32)
    pos = pos.astype(jnp.int32)
    neg = neg.astype(jnp.int32)

    def lid(r):
        return ((r >> 13) << 10) | (r & 1023)

    item_lines = _tc_repack(item_table.T)
    plines, nlines = _sc_gather(
        item_lines, [lid(pos).reshape(-1), lid(neg).reshape(-1)])
    user_lines = _tc_repack(user_table.T)
    (ulines,) = _sc_gather(user_lines, [lid(user).reshape(-1)])

    weights = (
        jnp.tile(uW0, (PACK, 1)), ub0.reshape(1, -1),
        uW1, ub1.reshape(1, -1), uW2, ub2.reshape(1, -1), uW3, ub3.reshape(1, -1),
        jnp.tile(iW0, (PACK, 1)), ib0.reshape(1, -1),
        iW1, ib1.reshape(1, -1), iW2, ib2.reshape(1, -1), iW3, ib3.reshape(1, -1),
        dW.reshape(1, -1), db.reshape(1, 1),
    )
    return _tc_mlp(user, pos, neg, ulines, plines, nlines, weights)


# speedup vs baseline: 2.8127x; 1.1321x over previous
import functools

import jax
import jax.numpy as jnp
from jax import lax
from jax.experimental import pallas as pl
from jax.experimental.pallas import tpu as pltpu
from jax.experimental.pallas import tpu_sc as plsc

B = 16384
VOCAB = 1000000
EMBED = 16
PACK = 128 // EMBED
CB = 8192
NBLK_RP = -(-VOCAB // CB)
NLINE = NBLK_RP * (CB // PACK)
NC, NS = 2, 16
NW = NC * NS
RPW = B // NW
CHUNK = 128
NCH = RPW // CHUNK
RB = 2048
NBLK = B // RB


def _repack_body(src, dst):
    LB = CB // PACK
    x = src[...]
    stacked = jnp.concatenate(
        [x[:, rw * LB:(rw + 1) * LB] for rw in range(PACK)], axis=0)
    dst[...] = jnp.concatenate(
        [stacked[:, l * 128:(l + 1) * 128].T for l in range(LB // 128)], axis=0)


def _tc_repack(tab_t):
    return pl.pallas_call(
        _repack_body,
        grid=(NBLK_RP,),
        in_specs=[pl.BlockSpec((EMBED, CB), lambda g: (0, g))],
        out_specs=pl.BlockSpec((CB // PACK, 128), lambda g: (g, 0)),
        out_shape=jax.ShapeDtypeStruct((NLINE, 128), jnp.float32),
    )(tab_t)


def _sc_gather(lines, idxs):
    n = len(idxs)
    mesh = plsc.VectorSubcoreMesh(core_axis_name="c", subcore_axis_name="s")
    out = jax.ShapeDtypeStruct((B, 128), jnp.float32)

    @functools.partial(
        pl.kernel,
        out_type=(out,) * n,
        mesh=mesh,
        scratch_types=[pltpu.VMEM((NCH, CHUNK), jnp.int32)] * n + [
            pltpu.VMEM((2, CHUNK, 128), jnp.float32),
            pltpu.SemaphoreType.DMA,
            pltpu.SemaphoreType.DMA,
        ],
    )
    def gather(tab, *refs):
        idx_hbm = refs[:n]
        outs = refs[n:2 * n]
        idx_v = refs[2 * n:3 * n]
        rows_v, gsem, wsem = refs[3 * n:]
        wid = lax.axis_index("s") * NC + lax.axis_index("c")
        base = wid * RPW
        for src, iv in zip(idx_hbm, idx_v):
            for j in range(NCH):
                pltpu.sync_copy(src.at[pl.ds(base + j * CHUNK, CHUNK)], iv.at[j])
        jobs = []
        for iv, dst in zip(idx_v, outs):
            for j in range(NCH):
                jobs.append((iv.at[j], dst.at[pl.ds(base + j * CHUNK, CHUNK)]))
        gd = [None] * len(jobs)
        wd = [None] * len(jobs)
        for c, (ivr, dst) in enumerate(jobs):
            if c >= 2:
                wd[c - 2].wait()
            gd[c] = pltpu.async_copy(tab.at[ivr], rows_v.at[c % 2], gsem)
            if c >= 1:
                gd[c - 1].wait()
                wd[c - 1] = pltpu.async_copy(rows_v.at[(c - 1) % 2], jobs[c - 1][1], wsem)
        last = len(jobs) - 1
        gd[last].wait()
        wd[last] = pltpu.async_copy(rows_v.at[last % 2], jobs[last][1], wsem)
        wd[last - 1].wait()
        wd[last].wait()

    return gather(lines, *idxs)


def _mlp_body(uid_ref, pid_ref, nid_ref, ul_ref, pl_ref, nl_ref,
              uw0, ub0, uw1, ub1, uw2, ub2, uw3, ub3,
              iw0, ib0, iw1, ib1, iw2, ib2, iw3, ib3,
              dw, db, out_ref):
    def mm(x, W):
        return jnp.dot(x, W, preferred_element_type=jnp.float32,
                       precision=lax.Precision.HIGHEST)

    lane_grp = lax.broadcasted_iota(jnp.int32, (RB, 128), 1) // EMBED
    def masked(lines_ref, id_ref):
        sub = (id_ref[...] >> 10) & (PACK - 1)
        return jnp.where(lane_grp == sub, lines_ref[...], 0.0)

    u = jnp.maximum(mm(masked(ul_ref, uid_ref), uw0[...]) + ub0[...], 0.0)
    p = jnp.maximum(mm(masked(pl_ref, pid_ref), iw0[...]) + ib0[...], 0.0)
    n = jnp.maximum(mm(masked(nl_ref, nid_ref), iw0[...]) + ib0[...], 0.0)
    for W, b in ((uw1, ub1), (uw2, ub2), (uw3, ub3)):
        u = jnp.maximum(mm(u, W[...]) + b[...], 0.0)
    for W, b in ((iw1, ib1), (iw2, ib2), (iw3, ib3)):
        Wv, bv = W[...], b[...]
        p = jnp.maximum(mm(p, Wv) + bv, 0.0)
        n = jnp.maximum(mm(n, Wv) + bv, 0.0)
    w = dw[...]
    bias = db[...]
    pv = jax.nn.sigmoid(u * p)
    nv = jax.nn.sigmoid(u * n)
    pos_l = jnp.sum(pv * w, axis=1, keepdims=True) + bias
    neg_l = jnp.sum(nv * w, axis=1, keepdims=True) + bias
    out_ref[...] = jnp.concatenate([pos_l, neg_l], axis=1)


def _tc_mlp(user, pos, neg, ul, plines, nl, weights):
    def wspec(w):
        return pl.BlockSpec(w.shape, lambda i: (0, 0))

    in_specs = [
        pl.BlockSpec((RB, 1), lambda i: (i, 0)),
        pl.BlockSpec((RB, 1), lambda i: (i, 0)),
        pl.BlockSpec((RB, 1), lambda i: (i, 0)),
        pl.BlockSpec((RB, 128), lambda i: (i, 0)),
        pl.BlockSpec((RB, 128), lambda i: (i, 0)),
        pl.BlockSpec((RB, 128), lambda i: (i, 0)),
    ] + [wspec(w) for w in weights]

    return pl.pallas_call(
        _mlp_body,
        grid=(NBLK,),
        in_specs=in_specs,
        out_specs=pl.BlockSpec((RB, 2), lambda i: (i, 0)),
        out_shape=jax.ShapeDtypeStruct((B, 2), jnp.float32),
    )(user, pos, neg, ul, plines, nl, *weights)


def kernel(user, pos, neg, user_table, item_table,
           uW0, ub0, uW1, ub1, uW2, ub2, uW3, ub3,
           iW0, ib0, iW1, ib1, iW2, ib2, iW3, ib3,
           dW, db):
    user = user.astype(jnp.int32)
    pos = pos.astype(jnp.int32)
    neg = neg.astype(jnp.int32)

    def lid(r):
        return ((r >> 13) << 10) | (r & 1023)

    item_lines = _tc_repack(item_table.T)
    plines, nlines = _sc_gather(
        item_lines, [lid(pos).reshape(-1), lid(neg).reshape(-1)])
    user_lines = _tc_repack(user_table.T)
    (ulines,) = _sc_gather(user_lines, [lid(user).reshape(-1)])

    weights = (
        jnp.tile(uW0, (PACK, 1)), ub0.reshape(1, -1),
        uW1, ub1.reshape(1, -1), uW2, ub2.reshape(1, -1), uW3, ub3.reshape(1, -1),
        jnp.tile(iW0, (PACK, 1)), ib0.reshape(1, -1),
        iW1, ib1.reshape(1, -1), iW2, ib2.reshape(1, -1), iW3, ib3.reshape(1, -1),
        dW.reshape(1, -1), db.reshape(1, 1),
    )
    return _tc_mlp(user, pos, neg, ulines, plines, nlines, weights)
